# baseline (speedup 1.0000x reference)
import functools

import jax
import jax.numpy as jnp
from jax import lax
from jax.experimental import pallas as pl
from jax.experimental.pallas import tpu as pltpu
from jax.experimental.pallas import tpu_sc as plsc

B = 16384
N_USER = 1000000
D_U = 32
N_CAT = 1000
D_C = 16
D_OUT = D_U + D_C + 2

NC = 2
NS = 16
NW = NC * NS
BPW = B // NW
CHUNK = 128
NCH = BPW // CHUNK
L = 16


def _sc_body(uid_hbm, cid_hbm, ds_hbm, dc_hbm, wu_hbm, wc_hbm, out_hbm,
             uid_v, cid_v, ug_v, cg_v, wc_v, ds_v, dc_v,
             sem_u, sem_o):
    wid = lax.axis_index("s") * NC + lax.axis_index("c")
    base = wid * BPW

    pltpu.sync_copy(uid_hbm.at[pl.ds(wid * NCH, NCH)], uid_v)
    pltpu.sync_copy(cid_hbm.at[pl.ds(wid * NCH, NCH)], cid_v)

    ucopies = []
    for j in range(D_U):
        col = wu_hbm.at[pl.ds(j * N_USER, N_USER)]
        for c in range(NCH):
            ucopies.append(pltpu.async_copy(
                col.at[uid_v.at[c]],
                ug_v.at[pl.ds((j * NCH + c) * CHUNK, CHUNK)],
                sem_u))

    pltpu.sync_copy(wc_hbm, wc_v)
    pltpu.sync_copy(ds_hbm.at[pl.ds(base, BPW)], ds_v)
    pltpu.sync_copy(dc_hbm.at[pl.ds(base, BPW)], dc_v)

    def cat_body(i, carry):
        j = i // (BPW // L)
        g = (i % (BPW // L)) * L
        idx = cid_v[g // CHUNK, pl.ds(g % CHUNK, L)] + j * N_CAT
        cg_v[pl.ds(j * BPW + g, L)] = plsc.load_gather(wc_v, [idx])
        return carry

    lax.fori_loop(0, D_C * (BPW // L), cat_body, 0, unroll=4)

    out_writes = []
    out_writes.append(pltpu.async_copy(
        ds_v, out_hbm.at[pl.ds((D_U + D_C) * B + base, BPW)], sem_o))
    out_writes.append(pltpu.async_copy(
        dc_v, out_hbm.at[pl.ds((D_U + D_C + 1) * B + base, BPW)], sem_o))
    for j in range(D_C):
        out_writes.append(pltpu.async_copy(
            cg_v.at[pl.ds(j * BPW, BPW)],
            out_hbm.at[pl.ds((D_U + j) * B + base, BPW)], sem_o))
    for j in range(D_U):
        for c in range(NCH):
            ucopies[j * NCH + c].wait()
        out_writes.append(pltpu.async_copy(
            ug_v.at[pl.ds(j * BPW, BPW)],
            out_hbm.at[pl.ds(j * B + base, BPW)], sem_o))
    for w in out_writes:
        w.wait()


def kernel(user_id, category, day_sin, day_cos, W_user, W_category):
    uid2 = user_id.reshape(B // CHUNK, CHUNK)
    cid2 = category.reshape(B // CHUNK, CHUNK)
    ds1 = day_sin.reshape(B)
    dc1 = day_cos.reshape(B)
    wuT = W_user.T.reshape(D_U * N_USER)
    wcT = W_category.T.reshape(D_C * N_CAT)
    mesh = plsc.VectorSubcoreMesh(core_axis_name="c", subcore_axis_name="s")
    run = functools.partial(
        pl.kernel, mesh=mesh,
        compiler_params=pltpu.CompilerParams(needs_layout_passes=False),
        out_type=jax.ShapeDtypeStruct((D_OUT * B,), jnp.float32),
        scratch_types=[
            pltpu.VMEM((NCH, CHUNK), jnp.int32),
            pltpu.VMEM((NCH, CHUNK), jnp.int32),
            pltpu.VMEM((D_U * BPW,), jnp.float32),
            pltpu.VMEM((D_C * BPW,), jnp.float32),
            pltpu.VMEM((D_C * N_CAT,), jnp.float32),
            pltpu.VMEM((BPW,), jnp.float32),
            pltpu.VMEM((BPW,), jnp.float32),
            pltpu.SemaphoreType.DMA,
            pltpu.SemaphoreType.DMA,
        ],
    )(_sc_body)
    flat = run(uid2, cid2, ds1, dc1, wuT, wcT)
    return flat.reshape(D_OUT, B).T


# speedup vs baseline: 3.5654x; 3.5654x over previous
import functools

import jax
import jax.numpy as jnp
from jax import lax
from jax.experimental import pallas as pl
from jax.experimental.pallas import tpu as pltpu
from jax.experimental.pallas import tpu_sc as plsc

B = 16384
N_USER = 1000000
D_U = 32
N_CAT = 1000
D_C = 16
D_OUT = D_U + D_C + 2

NC = 2
NS = 16
NW = NC * NS
BPW = B // NW
L = 16

SLAB = 2048
N_SLAB = N_USER // SLAB + 1
TAIL0 = 999936
MAXWB = 997888
OWN_CAP = 768
HIT_CAP = 128

SLABS_LO = N_SLAB // NW
EXTRA = N_SLAB - SLABS_LO * NW


def _sc_body(uid_hbm, cid_hbm, ds_hbm, dc_hbm, wu_hbm, wut_hbm, wc_hbm,
             out_hbm,
             uid_v, cid_v, own_u, own_b, slab_v, hitc_v, hitb_v, stage_v,
             tail_v, wc_v, cg_v, ds_v, dc_v, dump_v,
             sem_o, sem_s):
    wid = lax.axis_index("s") * NC + lax.axis_index("c")
    base = wid * BPW
    iota = lax.iota(jnp.int32, L)

    pltpu.sync_copy(wc_hbm, wc_v)
    pltpu.sync_copy(cid_hbm.at[pl.ds(base, BPW)], cid_v)
    pltpu.sync_copy(ds_hbm.at[pl.ds(base, BPW)], ds_v)
    pltpu.sync_copy(dc_hbm.at[pl.ds(base, BPW)], dc_v)

    def cat_body(i, carry):
        j = i // (BPW // L)
        g = (i % (BPW // L)) * L
        idx = cid_v[pl.ds(g, L)] + j * N_CAT
        cg_v[pl.ds(j * BPW + g, L)] = plsc.load_gather(wc_v, [idx])
        return carry

    lax.fori_loop(0, D_C * (BPW // L), cat_body, 0, unroll=4)

    out_writes = [
        pltpu.async_copy(ds_v, out_hbm.at[pl.ds((D_U + D_C) * B + base, BPW)],
                         sem_o),
        pltpu.async_copy(dc_v,
                         out_hbm.at[pl.ds((D_U + D_C + 1) * B + base, BPW)],
                         sem_o),
    ]
    for j in range(D_C):
        out_writes.append(pltpu.async_copy(
            cg_v.at[pl.ds(j * BPW, BPW)],
            out_hbm.at[pl.ds((D_U + j) * B + base, BPW)], sem_o))

    pltpu.sync_copy(uid_hbm, uid_v)
    start_w = SLABS_LO * wid + jnp.minimum(wid, EXTRA)
    count_w = SLABS_LO + jnp.where(wid < EXTRA, 1, 0)
    lo_w = start_w * SLAB
    hi_w = jnp.minimum((start_w + count_w) * SLAB, N_USER)

    def sel_body(g, cnt):
        uvec = uid_v[pl.ds(g * L, L)]
        m = (uvec >= lo_w) & (uvec < hi_w)
        cpos = jnp.minimum(cnt, OWN_CAP)
        plsc.store_compressed(own_u.at[pl.ds(cpos, L)], uvec, mask=m)
        plsc.store_compressed(own_b.at[pl.ds(cpos, L)], iota + g * L, mask=m)
        npos = plsc.all_reduce_population_count(m)
        return cnt + npos[0]

    cnt = lax.fori_loop(0, B // L, sel_body, jnp.int32(0), unroll=2)
    cnt = jnp.minimum(cnt, OWN_CAP)

    def hit_loop(nh, gather_pair):
        def hgroup(h, carry):
            off = h * L
            rem = nh - off
            cv = hitc_v[pl.ds(off, L)]
            bv = hitb_v[pl.ds(off, L)]
            keep = iota < rem
            cv = jnp.where(keep, cv, jnp.full((L,), cv[0], jnp.int32))
            bv = jnp.where(keep, bv, jnp.full((L,), bv[0], jnp.int32))
            for r in range(L):
                col = cv[r]
                v1, v2 = gather_pair(col)
                sw = (off + r) * D_U
                stage_v[pl.ds(sw, L)] = v1
                stage_v[pl.ds(sw + L, L)] = v2
                idx1 = iota * B + bv[r]
                pltpu.async_copy(stage_v.at[pl.ds(sw, L)],
                                 out_hbm.at[idx1], sem_s)
                pltpu.async_copy(stage_v.at[pl.ds(sw + L, L)],
                                 out_hbm.at[idx1 + L * B], sem_s)
            return carry

        ngroups = (nh + L - 1) // L
        lax.fori_loop(0, ngroups, hgroup, 0)

        def drain(h, carry):
            for _ in range(2 * L):
                pltpu.make_async_copy(
                    out_hbm.at[pl.ds(0, L)], dump_v, sem_s).wait()
            return carry

        lax.fori_loop(0, ngroups, drain, 0)

    def slab_body(i, carry):
        s = start_w + i
        lo_s = s * SLAB
        hi_s = jnp.minimum(lo_s + SLAB, TAIL0)
        wb = pl.multiple_of(jnp.minimum(lo_s, MAXWB), CHUNK)
        pltpu.sync_copy(wut_hbm.at[:, pl.ds(wb, SLAB)], slab_v)

        def rescan(g, nh):
            uvec = own_u[pl.ds(g * L, L)]
            bvec = own_b[pl.ds(g * L, L)]
            m = ((uvec >= lo_s) & (uvec < hi_s)
                 & ((iota + g * L) < cnt))
            hpos = jnp.minimum(nh, HIT_CAP)
            plsc.store_compressed(hitc_v.at[pl.ds(hpos, L)], uvec - wb, mask=m)
            plsc.store_compressed(hitb_v.at[pl.ds(hpos, L)], bvec, mask=m)
            return nh + plsc.all_reduce_population_count(m)[0]

        nown = (cnt + L - 1) // L
        nh = lax.fori_loop(0, nown, rescan, jnp.int32(0))
        nh = jnp.minimum(nh, HIT_CAP)

        def gather_slab(col):
            cs = jnp.full((L,), col, jnp.int32)
            return (plsc.load_gather(slab_v, [iota, cs]),
                    plsc.load_gather(slab_v, [iota + L, cs]))

        hit_loop(nh, gather_slab)
        return carry

    lax.fori_loop(0, count_w, slab_body, 0)

    @pl.when(wid == NW - 1)
    def _tail():
        pltpu.sync_copy(wu_hbm, tail_v)

        def rescan_t(g, nh):
            uvec = own_u[pl.ds(g * L, L)]
            bvec = own_b[pl.ds(g * L, L)]
            m = (uvec >= TAIL0) & ((iota + g * L) < cnt)
            hpos = jnp.minimum(nh, HIT_CAP)
            plsc.store_compressed(hitc_v.at[pl.ds(hpos, L)], uvec - TAIL0, mask=m)
            plsc.store_compressed(hitb_v.at[pl.ds(hpos, L)], bvec, mask=m)
            return nh + plsc.all_reduce_population_count(m)[0]

        nown = (cnt + L - 1) // L
        nh = lax.fori_loop(0, nown, rescan_t, jnp.int32(0))
        nh = jnp.minimum(nh, HIT_CAP)

        def gather_tail(col):
            idx = iota * (N_USER - TAIL0) + col
            return (plsc.load_gather(tail_v, [idx]),
                    plsc.load_gather(tail_v, [idx + L * (N_USER - TAIL0)]))

        hit_loop(nh, gather_tail)

    for w in out_writes:
        w.wait()


CHUNK = 128


def kernel(user_id, category, day_sin, day_cos, W_user, W_category):
    ds1 = day_sin.reshape(B)
    dc1 = day_cos.reshape(B)
    wuT = W_user.T
    wcT = W_category.T.reshape(D_C * N_CAT)
    wu_tail = W_user[TAIL0:].T.reshape(D_U * (N_USER - TAIL0))
    mesh = plsc.VectorSubcoreMesh(core_axis_name="c", subcore_axis_name="s")
    run = functools.partial(
        pl.kernel, mesh=mesh,
        compiler_params=pltpu.CompilerParams(needs_layout_passes=False),
        out_type=jax.ShapeDtypeStruct((D_OUT * B,), jnp.float32),
        scratch_types=[
            pltpu.VMEM((B,), jnp.int32),
            pltpu.VMEM((BPW,), jnp.int32),
            pltpu.VMEM((OWN_CAP + L,), jnp.int32),
            pltpu.VMEM((OWN_CAP + L,), jnp.int32),
            pltpu.VMEM((D_U, SLAB), jnp.float32),
            pltpu.VMEM((HIT_CAP + L,), jnp.int32),
            pltpu.VMEM((HIT_CAP + L,), jnp.int32),
            pltpu.VMEM(((HIT_CAP + L) * D_U,), jnp.float32),
            pltpu.VMEM((D_U * (N_USER - TAIL0),), jnp.float32),
            pltpu.VMEM((D_C * N_CAT,), jnp.float32),
            pltpu.VMEM((D_C * BPW,), jnp.float32),
            pltpu.VMEM((BPW,), jnp.float32),
            pltpu.VMEM((BPW,), jnp.float32),
            pltpu.VMEM((L,), jnp.float32),
            pltpu.SemaphoreType.DMA,
            pltpu.SemaphoreType.DMA,
        ],
    )(_sc_body)
    flat = run(user_id, category, ds1, dc1, wu_tail, wuT, wcT)
    return flat.reshape(D_OUT, B).T


# speedup vs baseline: 4.7586x; 1.3347x over previous
import functools

import jax
import jax.numpy as jnp
from jax import lax
from jax.experimental import pallas as pl
from jax.experimental.pallas import tpu as pltpu
from jax.experimental.pallas import tpu_sc as plsc

B = 16384
N_USER = 1000000
D_U = 32
N_CAT = 1000
D_C = 16
D_OUT = D_U + D_C + 2

NC = 2
NS = 16
NW = NC * NS
BPW = B // NW
CH = 128
NCH = BPW // CH
SLABW = 128
UPS = SLABW // D_U
L = 16


def _sc_body(uid_hbm, cid_hbm, wu_hbm, wc_hbm,
             outu_hbm, outc_hbm,
             uid_v, cid_v, qid_v, rows4_v, wc_v, outu_v, outc_v, sem_g):
    wid = lax.axis_index("s") * NC + lax.axis_index("c")
    base = wid * BPW

    pltpu.sync_copy(uid_hbm.at[pl.ds(base, BPW)], uid_v)
    pltpu.sync_copy(cid_hbm.at[pl.ds(base, BPW)], cid_v)
    cat_stage = pltpu.async_copy(wc_hbm, wc_v, sem_g)

    def qid_body(g, carry):
        u16 = uid_v[pl.ds(g * L, L)]
        qid_v[pl.ds(g * L, L)] = u16 >> UPS.bit_length() - 1
        return carry

    lax.fori_loop(0, BPW // L, qid_body, 0, unroll=8)

    gathers = []
    for j in range(NCH):
        gathers.append(pltpu.async_copy(
            wu_hbm.at[qid_v.at[pl.ds(j * CH, CH)]],
            rows4_v.at[pl.ds(j * CH, CH)], sem_g))

    cat_stage.wait()

    def cat_body(g, carry):
        c16 = cid_v[pl.ds(g * L, L)] * D_C
        for r in range(L):
            b = g * L + r
            outc_v[pl.ds(b * D_C, D_C)] = wc_v[pl.ds(c16[r], D_C)]
        return carry

    lax.fori_loop(0, BPW // L, cat_body, 0, unroll=2)

    for g in gathers:
        g.wait()

    def user_body(g, carry):
        o16 = (uid_v[pl.ds(g * L, L)] & (UPS - 1)) * D_U
        for r in range(L):
            b = g * L + r
            off = o16[r]
            outu_v[pl.ds(b * D_U, L)] = rows4_v[b, pl.ds(off, L)]
            outu_v[pl.ds(b * D_U + L, L)] = rows4_v[b, pl.ds(off + L, L)]
        return carry

    lax.fori_loop(0, BPW // L, user_body, 0, unroll=2)

    pltpu.sync_copy(outu_v, outu_hbm.at[pl.ds(base * D_U, BPW * D_U)])
    pltpu.sync_copy(outc_v, outc_hbm.at[pl.ds(base * D_C, BPW * D_C)])


def kernel(user_id, category, day_sin, day_cos, W_user, W_category):
    mesh = plsc.VectorSubcoreMesh(core_axis_name="c", subcore_axis_name="s")
    run = pl.kernel(
        _sc_body, mesh=mesh,
        out_type=(jax.ShapeDtypeStruct((B * D_U,), jnp.float32),
                  jax.ShapeDtypeStruct((B * D_C,), jnp.float32)),
        scratch_types=[
            pltpu.VMEM((BPW,), jnp.int32),
            pltpu.VMEM((BPW,), jnp.int32),
            pltpu.VMEM((BPW,), jnp.int32),
            pltpu.VMEM((BPW, SLABW), jnp.float32),
            pltpu.VMEM((N_CAT * D_C,), jnp.float32),
            pltpu.VMEM((BPW * D_U,), jnp.float32),
            pltpu.VMEM((BPW * D_C,), jnp.float32),
            pltpu.SemaphoreType.DMA,
        ],
    )
    eu, ec = run(user_id, category,
                 W_user.reshape(N_USER // UPS, SLABW),
                 W_category.reshape(N_CAT * D_C))
    return jnp.concatenate([eu.reshape(B, D_U), ec.reshape(B, D_C),
                            day_sin, day_cos], axis=1)
